# baseline (device time: 130423 ns/iter reference)
import functools

import jax
import jax.numpy as jnp
from jax import lax
from jax.experimental import pallas as pl
from jax.experimental.pallas import tpu as pltpu

M_SHARD = 8192
N_SHARD = 1024
Q = M_SHARD // 2

SIZES = (128, 256, 512, 512, 512, 512, 512, 512, 512, 128)
assert sum(SIZES) == Q
OFFS = tuple(sum(SIZES[:i]) for i in range(len(SIZES)))
K = len(SIZES)
CMAX = max(SIZES)

CL = 512
LK = M_SHARD // CL
BF16 = jnp.bfloat16


def kernel(x):
    m, n = x.shape
    assert (m, n) == (M_SHARD, 2 * N_SHARD), (m, n)

    def body(x_ref, out_ref,
             qf32, ysend, yrecv, locf32, locb16,
             qload_sems, ysend_sems, yrecv_sems, xsend_sems, xrecv_sems,
             ystore_sems, locload_sems, locstore_sems):
        my_x = lax.axis_index("x")
        my_y = lax.axis_index("y")
        other_x = 1 - my_x
        other_y = 1 - my_y

        def qload(i):
            return pltpu.make_async_copy(
                x_ref.at[pl.ds(my_x * Q + OFFS[i], SIZES[i]),
                         pl.ds(other_y * N_SHARD, N_SHARD)],
                qf32.at[i % 2, pl.ds(0, SIZES[i])], qload_sems.at[i % 2])

        def y_rdma(i):
            return pltpu.make_async_remote_copy(
                src_ref=ysend.at[pl.ds(OFFS[i], SIZES[i])],
                dst_ref=yrecv.at[pl.ds(OFFS[i], SIZES[i])],
                send_sem=ysend_sems.at[i], recv_sem=yrecv_sems.at[i],
                device_id=(my_x, other_y),
                device_id_type=pl.DeviceIdType.MESH)

        def x_send(i):
            return pltpu.make_async_remote_copy(
                src_ref=yrecv.at[pl.ds(OFFS[i], SIZES[i])],
                dst_ref=out_ref.at[
                    pl.ds(other_y * M_SHARD + my_x * Q + OFFS[i], SIZES[i]), :],
                send_sem=xsend_sems.at[i], recv_sem=xrecv_sems.at[i],
                device_id=(other_x, my_y),
                device_id_type=pl.DeviceIdType.MESH)

        def x_wait(i):
            return pltpu.make_async_remote_copy(
                src_ref=yrecv.at[pl.ds(OFFS[i], SIZES[i])],
                dst_ref=out_ref.at[
                    pl.ds(other_y * M_SHARD + other_x * Q + OFFS[i], SIZES[i]), :],
                send_sem=xsend_sems.at[i], recv_sem=xrecv_sems.at[i],
                device_id=(other_x, my_y),
                device_id_type=pl.DeviceIdType.MESH)

        def ystore(i):
            return pltpu.make_async_copy(
                yrecv.at[pl.ds(OFFS[i], SIZES[i])],
                out_ref.at[
                    pl.ds(other_y * M_SHARD + my_x * Q + OFFS[i], SIZES[i]), :],
                ystore_sems.at[i])

        def locload(j):
            return pltpu.make_async_copy(
                x_ref.at[pl.ds(j * CL, CL), pl.ds(my_y * N_SHARD, N_SHARD)],
                locf32.at[j % 2], locload_sems.at[j % 2])

        def locstore(j):
            return pltpu.make_async_copy(
                locb16.at[j % 2],
                out_ref.at[pl.ds(my_y * M_SHARD + j * CL, CL), :],
                locstore_sems.at[j % 2])

        qload(0).start()
        qload(1).start()

        barrier_sem = pltpu.get_barrier_semaphore()
        for dev in ((my_x, other_y), (other_x, my_y)):
            pl.semaphore_signal(barrier_sem, inc=1, device_id=dev,
                                device_id_type=pl.DeviceIdType.MESH)
        pl.semaphore_wait(barrier_sem, 2)

        for i in range(K):
            qload(i).wait()
            ysend[pl.ds(OFFS[i], SIZES[i]), :] = (
                qf32[i % 2, pl.ds(0, SIZES[i]), :].astype(BF16))
            if i + 2 < K:
                qload(i + 2).start()
            y_rdma(i).start()

        locload(0).start()
        j = 0
        for i in range(K):
            y_rdma(i).wait_recv()
            x_send(i).start()
            ystore(i).start()
            if i >= 1:
                x_wait(i - 1).wait_recv()
            for _ in range(2):
                if j < LK:
                    if j + 1 < LK:
                        locload(j + 1).start()
                    locload(j).wait()
                    if j >= 2:
                        locstore(j - 2).wait()
                    locb16[j % 2, :, :] = locf32[j % 2, :, :].astype(BF16)
                    locstore(j).start()
                    j += 1
        x_wait(K - 1).wait_recv()

        for i in range(K):
            y_rdma(i).wait_send()
            x_send(i).wait_send()
            ystore(i).wait()
        locstore(LK - 2).wait()
        locstore(LK - 1).wait()

        @functools.partial(pl.run_scoped,
                           second_barrier=pltpu.SemaphoreType.REGULAR)
        def _(second_barrier):
            for dev in ((my_x, other_y), (other_x, my_y)):
                pl.semaphore_signal(second_barrier, inc=1, device_id=dev,
                                    device_id_type=pl.DeviceIdType.MESH)
            pl.semaphore_wait(second_barrier, 2)

    return pl.pallas_call(
        body,
        out_shape=jax.ShapeDtypeStruct((2 * M_SHARD, N_SHARD), BF16),
        in_specs=[pl.BlockSpec(memory_space=pl.ANY)],
        out_specs=pl.BlockSpec(memory_space=pl.ANY),
        scratch_shapes=[
            pltpu.VMEM((2, CMAX, N_SHARD), jnp.float32),
            pltpu.VMEM((Q, N_SHARD), BF16),
            pltpu.VMEM((Q, N_SHARD), BF16),
            pltpu.VMEM((2, CL, N_SHARD), jnp.float32),
            pltpu.VMEM((2, CL, N_SHARD), BF16),
            pltpu.SemaphoreType.DMA((2,)),
            pltpu.SemaphoreType.DMA((K,)),
            pltpu.SemaphoreType.DMA((K,)),
            pltpu.SemaphoreType.DMA((K,)),
            pltpu.SemaphoreType.DMA((K,)),
            pltpu.SemaphoreType.DMA((K,)),
            pltpu.SemaphoreType.DMA((2,)),
            pltpu.SemaphoreType.DMA((2,)),
        ],
        compiler_params=pltpu.CompilerParams(collective_id=0),
    )(x)


# device time: 124409 ns/iter; 1.0483x vs baseline; 1.0483x over previous
import functools

import jax
import jax.numpy as jnp
from jax import lax
from jax.experimental import pallas as pl
from jax.experimental.pallas import tpu as pltpu

M_SHARD = 8192
N_SHARD = 1024
Q = M_SHARD // 2

SIZES = (64, 192) + (256,) * 14 + (64, 192)
assert sum(SIZES) == Q
OFFS = tuple(sum(SIZES[:i]) for i in range(len(SIZES)))
K = len(SIZES)
CMAX = max(SIZES)
E = 192
XK = K - 1
assert OFFS[XK] == Q - E
EXTRA = K
D = 2

CL = 512
LK = M_SHARD // CL
LSLOTS = 4
BF16 = jnp.bfloat16


def kernel(x):
    m, n = x.shape
    assert (m, n) == (M_SHARD, 2 * N_SHARD), (m, n)
    assert LK <= K

    def body(x_ref, out_ref,
             qf32, ysend, yrecv, locf32, locb16,
             qload_sems, ysend_sems, yrecv_sems, xsend_sems, xrecv_sems,
             ystore_sems, locload_sems, locstore_sems):
        my_x = lax.axis_index("x")
        my_y = lax.axis_index("y")
        other_x = 1 - my_x
        other_y = 1 - my_y

        def src_row(i):
            if i == EXTRA:
                return other_x * Q + (Q - E)
            return my_x * Q + OFFS[i]

        def slot_off(i):
            return Q if i == EXTRA else OFFS[i]

        def size(i):
            return E if i == EXTRA else SIZES[i]

        def dst_out_row_send(i):
            if i == EXTRA:
                return my_y * M_SHARD + my_x * Q + (Q - E)
            return my_y * M_SHARD + my_x * Q + OFFS[i]

        def dst_out_row_recv(i):
            if i == EXTRA:
                return other_y * M_SHARD + other_x * Q + (Q - E)
            return other_y * M_SHARD + my_x * Q + OFFS[i]

        def qload(i):
            return pltpu.make_async_copy(
                x_ref.at[pl.ds(src_row(i), size(i)),
                         pl.ds(other_y * N_SHARD, N_SHARD)],
                qf32.at[i % 2, pl.ds(0, size(i))], qload_sems.at[i % 2])

        def y_rdma(i):
            return pltpu.make_async_remote_copy(
                src_ref=ysend.at[pl.ds(slot_off(i), size(i))],
                dst_ref=yrecv.at[pl.ds(slot_off(i), size(i))],
                send_sem=ysend_sems.at[i], recv_sem=yrecv_sems.at[i],
                device_id=(my_x, other_y),
                device_id_type=pl.DeviceIdType.MESH)

        def x_send(i):
            return pltpu.make_async_remote_copy(
                src_ref=yrecv.at[pl.ds(OFFS[i], SIZES[i])],
                dst_ref=out_ref.at[
                    pl.ds(other_y * M_SHARD + my_x * Q + OFFS[i], SIZES[i]), :],
                send_sem=xsend_sems.at[i], recv_sem=xrecv_sems.at[i],
                device_id=(other_x, my_y),
                device_id_type=pl.DeviceIdType.MESH)

        def x_wait(i):
            return pltpu.make_async_remote_copy(
                src_ref=yrecv.at[pl.ds(OFFS[i], SIZES[i])],
                dst_ref=out_ref.at[
                    pl.ds(other_y * M_SHARD + other_x * Q + OFFS[i], SIZES[i]), :],
                send_sem=xsend_sems.at[i], recv_sem=xrecv_sems.at[i],
                device_id=(other_x, my_y),
                device_id_type=pl.DeviceIdType.MESH)

        def ystore(i):
            return pltpu.make_async_copy(
                yrecv.at[pl.ds(slot_off(i), size(i))],
                out_ref.at[pl.ds(dst_out_row_recv(i), size(i)), :],
                ystore_sems.at[i])

        def locload(j):
            return pltpu.make_async_copy(
                x_ref.at[pl.ds(j * CL, CL), pl.ds(my_y * N_SHARD, N_SHARD)],
                locf32.at[j % LSLOTS], locload_sems.at[j % LSLOTS])

        def locstore(j):
            return pltpu.make_async_copy(
                locb16.at[j % 2],
                out_ref.at[pl.ds(my_y * M_SHARD + j * CL, CL), :],
                locstore_sems.at[j % 2])

        def fwd(i):
            y_rdma(i).wait_recv()
            if i < XK:
                x_send(i).start()
            ystore(i).start()

        qload(0).start()
        qload(1).start()
        for j in range(LSLOTS):
            locload(j).start()

        barrier_sem = pltpu.get_barrier_semaphore()
        for dev in ((my_x, other_y), (other_x, my_y)):
            pl.semaphore_signal(barrier_sem, inc=1, device_id=dev,
                                device_id_type=pl.DeviceIdType.MESH)
        pl.semaphore_wait(barrier_sem, 2)

        for i in range(K + 1):
            qload(i).wait()
            ysend[pl.ds(slot_off(i), size(i)), :] = (
                qf32[i % 2, pl.ds(0, size(i)), :].astype(BF16))
            if i + 2 < K + 1:
                qload(i + 2).start()
            y_rdma(i).start()
            if i >= D:
                fwd(i - D)
            if i < LK:
                locload(i).wait()
                if i >= 2:
                    locstore(i - 2).wait()
                locb16[i % 2, :, :] = locf32[i % LSLOTS, :, :].astype(BF16)
                locstore(i).start()
                if i + LSLOTS < LK:
                    locload(i + LSLOTS).start()

        for i in range(K + 1 - D, K + 1):
            fwd(i)

        for i in range(XK):
            x_wait(i).wait_recv()
        for i in range(K + 1):
            y_rdma(i).wait_send()
            ystore(i).wait()
        for i in range(XK):
            x_send(i).wait_send()
        locstore(LK - 2).wait()
        locstore(LK - 1).wait()

        @functools.partial(pl.run_scoped,
                           second_barrier=pltpu.SemaphoreType.REGULAR)
        def _(second_barrier):
            for dev in ((my_x, other_y), (other_x, my_y)):
                pl.semaphore_signal(second_barrier, inc=1, device_id=dev,
                                    device_id_type=pl.DeviceIdType.MESH)
            pl.semaphore_wait(second_barrier, 2)

    return pl.pallas_call(
        body,
        out_shape=jax.ShapeDtypeStruct((2 * M_SHARD, N_SHARD), BF16),
        in_specs=[pl.BlockSpec(memory_space=pl.ANY)],
        out_specs=pl.BlockSpec(memory_space=pl.ANY),
        scratch_shapes=[
            pltpu.VMEM((2, CMAX, N_SHARD), jnp.float32),
            pltpu.VMEM((Q + E, N_SHARD), BF16),
            pltpu.VMEM((Q + E, N_SHARD), BF16),
            pltpu.VMEM((LSLOTS, CL, N_SHARD), jnp.float32),
            pltpu.VMEM((2, CL, N_SHARD), BF16),
            pltpu.SemaphoreType.DMA((2,)),
            pltpu.SemaphoreType.DMA((K + 1,)),
            pltpu.SemaphoreType.DMA((K + 1,)),
            pltpu.SemaphoreType.DMA((XK,)),
            pltpu.SemaphoreType.DMA((XK,)),
            pltpu.SemaphoreType.DMA((K + 1,)),
            pltpu.SemaphoreType.DMA((LSLOTS,)),
            pltpu.SemaphoreType.DMA((2,)),
        ],
        compiler_params=pltpu.CompilerParams(collective_id=0),
    )(x)
